# baseline (speedup 1.0000x reference)
import functools

import jax
import jax.numpy as jnp
import numpy as np
from jax.experimental import pallas as pl
from jax.experimental.pallas import tpu as pltpu

K_NN = 100
EPS = 1e-12
PAD_VAL = 1e30
PAD_BITS = int(np.float32(PAD_VAL).view(np.int32))
_D2_PREC = jax.lax.Precision.DEFAULT


def _select_kernel(x_ref, xkt_ref, tau_ref, h2_ref, bits_ref, *, n_real, n_pad, ck, qb):
    j = pl.program_id(1)
    nchunks = n_pad // ck
    xb = x_ref[...]
    ab = jax.lax.dot_general(
        xb, xkt_ref[...], (((1,), (0,)), ((), ())),
        precision=_D2_PREC,
        preferred_element_type=jnp.float32,
    )
    xsq = jnp.sum(xb * xb, axis=1, keepdims=True)
    ksq = jnp.sum(xkt_ref[...] * xkt_ref[...], axis=0, keepdims=True)
    d2 = xsq + ksq - 2.0 * ab
    col = j * ck + jax.lax.broadcasted_iota(jnp.int32, (qb, ck), 1)
    d2 = jnp.where(col >= n_real, PAD_VAL, d2)
    d2c = jnp.maximum(d2, EPS)
    bits_ref[:, pl.ds(j * ck, ck)] = jax.lax.bitcast_convert_type(d2c, jnp.int32)

    @pl.when(j == nchunks - 1)
    def _finish():
        pad_bits = jnp.int32(PAD_BITS)

        def count_le(mid):
            def body(c, acc):
                blk = bits_ref[:, pl.ds(c * ck, ck)]
                return acc + jnp.sum(
                    (blk <= mid).astype(jnp.int32), axis=1, keepdims=True
                )
            return jax.lax.fori_loop(0, nchunks, body, jnp.zeros((qb, 1), jnp.int32))

        def minmax(c, carry):
            mn, mx = carry
            blk = bits_ref[:, pl.ds(c * ck, ck)]
            real = jnp.where(blk == pad_bits, jnp.int32(0), blk)
            return (
                jnp.minimum(mn, jnp.min(blk, axis=1, keepdims=True)),
                jnp.maximum(mx, jnp.max(real, axis=1, keepdims=True)),
            )

        big = jnp.full((qb, 1), jnp.int32(0x7F000000))
        mn, mx = jax.lax.fori_loop(0, nchunks, minmax, (big, -big))

        lo = mn - 1
        clo = jnp.zeros((qb, 1), jnp.int32)
        hi = mx
        chi = jnp.full((qb, 1), jnp.int32(n_real))

        def done(lo, hi, chi):
            return (chi == K_NN) | (hi - lo <= 1)

        def cond(carry):
            lo, hi, clo, chi, k = carry
            return jnp.any(~done(lo, hi, chi)) & (k < 48)

        def body(carry):
            lo, hi, clo, chi, k = carry
            d = done(lo, hi, chi)
            lof = lo.astype(jnp.float32)
            hif = hi.astype(jnp.float32)
            t = (jnp.log(jnp.float32(K_NN)) - jnp.log(clo.astype(jnp.float32) + 1.0)) / (
                jnp.log(chi.astype(jnp.float32) + 1.0)
                - jnp.log(clo.astype(jnp.float32) + 1.0)
            )
            m_interp = jnp.clip(
                (lof + (hif - lof) * t).astype(jnp.int32), lo + 1, hi - 1
            )
            m_bisect = lo + jax.lax.shift_right_logical(hi - lo, 1)
            m = jnp.where(d, hi, jnp.where(k < 10, m_interp, m_bisect))
            c = count_le(m)
            ge = c >= K_NN
            new_hi = jnp.where(d, hi, jnp.where(ge, m, hi))
            new_chi = jnp.where(d, chi, jnp.where(ge, c, chi))
            new_lo = jnp.where(d, lo, jnp.where(ge, lo, m))
            new_clo = jnp.where(d, clo, jnp.where(ge, clo, c))
            return new_lo, new_hi, new_clo, new_chi, k + 1

        lo, hi, clo, chi, _ = jax.lax.while_loop(
            cond, body, (lo, hi, clo, chi, jnp.int32(0))
        )
        tau_bits = hi

        def maxle(c, acc):
            blk = bits_ref[:, pl.ds(c * ck, ck)]
            sel = jnp.where(blk <= tau_bits, blk, jnp.int32(0))
            return jnp.maximum(acc, jnp.max(sel, axis=1, keepdims=True))

        h2_bits = jax.lax.fori_loop(0, nchunks, maxle, jnp.zeros((qb, 1), jnp.int32))
        tau_ref[...] = jax.lax.bitcast_convert_type(tau_bits, jnp.float32)
        h2_ref[...] = jax.lax.bitcast_convert_type(h2_bits, jnp.float32)


def _combine_kernel(x_ref, xkt_ref, vk_ref, tau_ref, h2_ref, w1_ref, c1_ref,
                    w2_ref, b2_ref, xdot_ref, cos_ref, l2_ref, u_ref, s_ref,
                    *, n_real, n_pad, ck, qb):
    j = pl.program_id(1)
    nchunks = n_pad // ck

    @pl.when(j == 0)
    def _init():
        u_ref[...] = jnp.zeros_like(u_ref)
        s_ref[...] = jnp.zeros_like(s_ref)

    xb = x_ref[...]
    ab = jax.lax.dot_general(
        xb, xkt_ref[...], (((1,), (0,)), ((), ())),
        precision=_D2_PREC,
        preferred_element_type=jnp.float32,
    )
    xsq = jnp.sum(xb * xb, axis=1, keepdims=True)
    ksq = jnp.sum(xkt_ref[...] * xkt_ref[...], axis=0, keepdims=True)
    d2 = xsq + ksq - 2.0 * ab
    col = j * ck + jax.lax.broadcasted_iota(jnp.int32, (qb, ck), 1)
    d2 = jnp.where(col >= n_real, PAD_VAL, d2)
    d2c = jnp.maximum(d2, EPS)

    tau = tau_ref[...]
    h2 = h2_ref[...]
    w = jnp.where(d2c <= tau, jnp.exp(-d2c / (2.0 * h2)), 0.0)
    s_ref[...] += jnp.sum(w, axis=1, keepdims=True)
    u_ref[...] += jax.lax.dot_general(
        w, vk_ref[...], (((1,), (0,)), ((), ())),
        precision=jax.lax.Precision.HIGHEST,
        preferred_element_type=jnp.float32,
    )

    @pl.when(j == nchunks - 1)
    def _finish():
        u_t = u_ref[...] / (s_ref[...] + EPS)
        h = jnp.tanh(
            jax.lax.dot_general(
                xb, w1_ref[...], (((1,), (0,)), ((), ())),
                precision=jax.lax.Precision.HIGHEST,
                preferred_element_type=jnp.float32,
            )
            + c1_ref[...]
        )
        x_dot = (
            jax.lax.dot_general(
                h, w2_ref[...], (((1,), (0,)), ((), ())),
                precision=jax.lax.Precision.HIGHEST,
                preferred_element_type=jnp.float32,
            )
            + b2_ref[...]
        )
        num = jnp.sum(u_t * x_dot, axis=1, keepdims=True)
        nu = jnp.sqrt(jnp.sum(u_t * u_t, axis=1, keepdims=True))
        nx = jnp.sqrt(jnp.sum(x_dot * x_dot, axis=1, keepdims=True))
        den = jnp.maximum(nu, 1e-8) * jnp.maximum(nx, 1e-8)
        diff = u_t - x_dot
        xdot_ref[...] = x_dot
        cos_ref[...] = 1.0 - num / den
        l2_ref[...] = jnp.sum(diff * diff, axis=1, keepdims=True)


@jax.jit
def kernel(t, z, x0, x1, v0, v1, W1, b1, W2, b2):
    x = z[:, :-3]
    B, D = x.shape
    n_real = x0.shape[0] + x1.shape[0]
    ck = min(4096, max(512, n_real))
    n_pad = ((n_real + ck - 1) // ck) * ck
    qb = min(64, B)
    nchunks = n_pad // ck
    nqb = B // qb

    xk = jnp.concatenate([x0, x1], axis=0)
    vk = jnp.concatenate([v0, v1], axis=0)
    xkt = jnp.pad(xk.T, ((0, 0), (0, n_pad - n_real)))
    vkp = jnp.pad(vk, ((0, n_pad - n_real), (0, 0)))

    H = W1.shape[1]
    W1a = W1[:D]
    c1 = (t[0] * W1[D] + b1).reshape(1, H)
    b2r = b2.reshape(1, D)

    tau, h2 = pl.pallas_call(
        functools.partial(_select_kernel, n_real=n_real, n_pad=n_pad, ck=ck, qb=qb),
        grid=(nqb, nchunks),
        in_specs=[
            pl.BlockSpec((qb, D), lambda i, j: (i, 0)),
            pl.BlockSpec((D, ck), lambda i, j: (0, j)),
        ],
        out_specs=[
            pl.BlockSpec((qb, 1), lambda i, j: (i, 0)),
            pl.BlockSpec((qb, 1), lambda i, j: (i, 0)),
        ],
        out_shape=[
            jax.ShapeDtypeStruct((B, 1), jnp.float32),
            jax.ShapeDtypeStruct((B, 1), jnp.float32),
        ],
        scratch_shapes=[pltpu.VMEM((qb, n_pad), jnp.int32)],
        compiler_params=pltpu.CompilerParams(
            dimension_semantics=("parallel", "arbitrary"),
        ),
    )(x, xkt)

    xdot, cos, l2 = pl.pallas_call(
        functools.partial(_combine_kernel, n_real=n_real, n_pad=n_pad, ck=ck, qb=qb),
        grid=(nqb, nchunks),
        in_specs=[
            pl.BlockSpec((qb, D), lambda i, j: (i, 0)),
            pl.BlockSpec((D, ck), lambda i, j: (0, j)),
            pl.BlockSpec((ck, D), lambda i, j: (j, 0)),
            pl.BlockSpec((qb, 1), lambda i, j: (i, 0)),
            pl.BlockSpec((qb, 1), lambda i, j: (i, 0)),
            pl.BlockSpec((D, H), lambda i, j: (0, 0)),
            pl.BlockSpec((1, H), lambda i, j: (0, 0)),
            pl.BlockSpec((H, D), lambda i, j: (0, 0)),
            pl.BlockSpec((1, D), lambda i, j: (0, 0)),
        ],
        out_specs=[
            pl.BlockSpec((qb, D), lambda i, j: (i, 0)),
            pl.BlockSpec((qb, 1), lambda i, j: (i, 0)),
            pl.BlockSpec((qb, 1), lambda i, j: (i, 0)),
        ],
        out_shape=[
            jax.ShapeDtypeStruct((B, D), jnp.float32),
            jax.ShapeDtypeStruct((B, 1), jnp.float32),
            jax.ShapeDtypeStruct((B, 1), jnp.float32),
        ],
        scratch_shapes=[
            pltpu.VMEM((qb, D), jnp.float32),
            pltpu.VMEM((qb, 1), jnp.float32),
        ],
        compiler_params=pltpu.CompilerParams(
            dimension_semantics=("parallel", "arbitrary"),
        ),
    )(x, xkt, vkp, tau, h2, W1a, c1, W2, b2r)

    return jnp.concatenate([xdot, cos, cos, l2], axis=1)


# speedup vs baseline: 1.4215x; 1.4215x over previous
import functools

import jax
import jax.numpy as jnp
import numpy as np
from jax.experimental import pallas as pl
from jax.experimental.pallas import tpu as pltpu

K_NN = 100
EPS = 1e-12
PAD_VAL = 1e30
PAD_BITS = int(np.float32(PAD_VAL).view(np.int32))
_D2_PREC = jax.lax.Precision.DEFAULT


def _select_kernel(x_ref, xkt_ref, tau_ref, h2_ref, bits_ref, *, n_real, n_pad, ck, qb):
    j = pl.program_id(1)
    nchunks = n_pad // ck
    xb = x_ref[...]
    ab = jax.lax.dot_general(
        xb, xkt_ref[...], (((1,), (0,)), ((), ())),
        precision=_D2_PREC,
        preferred_element_type=jnp.float32,
    )
    xsq = jnp.sum(xb * xb, axis=1, keepdims=True)
    ksq = jnp.sum(xkt_ref[...] * xkt_ref[...], axis=0, keepdims=True)
    d2 = xsq + ksq - 2.0 * ab
    col = j * ck + jax.lax.broadcasted_iota(jnp.int32, (qb, ck), 1)
    d2 = jnp.where(col >= n_real, PAD_VAL, d2)
    d2c = jnp.maximum(d2, EPS)
    bits_ref[:, pl.ds(j * ck, ck)] = jax.lax.bitcast_convert_type(d2c, jnp.int32)

    @pl.when(j == nchunks - 1)
    def _finish():
        pad_bits = jnp.int32(PAD_BITS)

        def count_le(mid):
            def body(c, acc):
                blk = bits_ref[:, pl.ds(c * ck, ck)]
                return acc + jnp.sum(
                    (blk <= mid).astype(jnp.int32), axis=1, keepdims=True
                )
            return jax.lax.fori_loop(0, nchunks, body, jnp.zeros((qb, 1), jnp.int32))

        def minmax(c, carry):
            mn, mx = carry
            blk = bits_ref[:, pl.ds(c * ck, ck)]
            real = jnp.where(blk == pad_bits, jnp.int32(0), blk)
            return (
                jnp.minimum(mn, jnp.min(blk, axis=1, keepdims=True)),
                jnp.maximum(mx, jnp.max(real, axis=1, keepdims=True)),
            )

        big = jnp.full((qb, 1), jnp.int32(0x7F000000))
        mn, mx = jax.lax.fori_loop(0, nchunks, minmax, (big, -big))

        lo = mn - 1
        clo = jnp.zeros((qb, 1), jnp.int32)
        hi = mx
        chi = jnp.full((qb, 1), jnp.int32(n_real))

        def done(lo, hi, chi):
            return (chi == K_NN) | (hi - lo <= 1)

        def cond(carry):
            lo, hi, clo, chi, k = carry
            return jnp.any(~done(lo, hi, chi)) & (k < 48)

        def body(carry):
            lo, hi, clo, chi, k = carry
            d = done(lo, hi, chi)
            lof = lo.astype(jnp.float32)
            hif = hi.astype(jnp.float32)
            t = (jnp.log(jnp.float32(K_NN)) - jnp.log(clo.astype(jnp.float32) + 1.0)) / (
                jnp.log(chi.astype(jnp.float32) + 1.0)
                - jnp.log(clo.astype(jnp.float32) + 1.0)
            )
            m_interp = jnp.clip(
                (lof + (hif - lof) * t).astype(jnp.int32), lo + 1, hi - 1
            )
            m_bisect = lo + jax.lax.shift_right_logical(hi - lo, 1)
            m = jnp.where(d, hi, jnp.where(k < 10, m_interp, m_bisect))
            c = count_le(m)
            ge = c >= K_NN
            new_hi = jnp.where(d, hi, jnp.where(ge, m, hi))
            new_chi = jnp.where(d, chi, jnp.where(ge, c, chi))
            new_lo = jnp.where(d, lo, jnp.where(ge, lo, m))
            new_clo = jnp.where(d, clo, jnp.where(ge, clo, c))
            return new_lo, new_hi, new_clo, new_chi, k + 1

        lo, hi, clo, chi, _ = jax.lax.while_loop(
            cond, body, (lo, hi, clo, chi, jnp.int32(0))
        )
        tau_bits = hi

        def maxle(c, acc):
            blk = bits_ref[:, pl.ds(c * ck, ck)]
            sel = jnp.where(blk <= tau_bits, blk, jnp.int32(0))
            return jnp.maximum(acc, jnp.max(sel, axis=1, keepdims=True))

        h2_bits = jax.lax.fori_loop(0, nchunks, maxle, jnp.zeros((qb, 1), jnp.int32))
        tau_ref[...] = jax.lax.bitcast_convert_type(tau_bits, jnp.float32)
        h2_ref[...] = jax.lax.bitcast_convert_type(h2_bits, jnp.float32)


def _combine_kernel(x_ref, xkt_ref, vk_ref, tau_ref, h2_ref, w1_ref, c1_ref,
                    w2_ref, b2_ref, xdot_ref, cos_ref, l2_ref, u_ref, s_ref,
                    *, n_real, n_pad, ck, qb):
    j = pl.program_id(1)
    nchunks = n_pad // ck

    @pl.when(j == 0)
    def _init():
        u_ref[...] = jnp.zeros_like(u_ref)
        s_ref[...] = jnp.zeros_like(s_ref)

    xb = x_ref[...]
    ab = jax.lax.dot_general(
        xb, xkt_ref[...], (((1,), (0,)), ((), ())),
        precision=_D2_PREC,
        preferred_element_type=jnp.float32,
    )
    xsq = jnp.sum(xb * xb, axis=1, keepdims=True)
    ksq = jnp.sum(xkt_ref[...] * xkt_ref[...], axis=0, keepdims=True)
    d2 = xsq + ksq - 2.0 * ab
    col = j * ck + jax.lax.broadcasted_iota(jnp.int32, (qb, ck), 1)
    d2 = jnp.where(col >= n_real, PAD_VAL, d2)
    d2c = jnp.maximum(d2, EPS)

    tau = tau_ref[...]
    h2 = h2_ref[...]
    w = jnp.where(d2c <= tau, jnp.exp(-d2c / (2.0 * h2)), 0.0)
    s_ref[...] += jnp.sum(w, axis=1, keepdims=True)
    u_ref[...] += jax.lax.dot_general(
        w, vk_ref[...], (((1,), (0,)), ((), ())),
        precision=jax.lax.Precision.DEFAULT,
        preferred_element_type=jnp.float32,
    )

    @pl.when(j == nchunks - 1)
    def _finish():
        u_t = u_ref[...] / (s_ref[...] + EPS)
        h = jnp.tanh(
            jax.lax.dot_general(
                xb, w1_ref[...], (((1,), (0,)), ((), ())),
                precision=jax.lax.Precision.DEFAULT,
                preferred_element_type=jnp.float32,
            )
            + c1_ref[...]
        )
        x_dot = (
            jax.lax.dot_general(
                h, w2_ref[...], (((1,), (0,)), ((), ())),
                precision=jax.lax.Precision.DEFAULT,
                preferred_element_type=jnp.float32,
            )
            + b2_ref[...]
        )
        num = jnp.sum(u_t * x_dot, axis=1, keepdims=True)
        nu = jnp.sqrt(jnp.sum(u_t * u_t, axis=1, keepdims=True))
        nx = jnp.sqrt(jnp.sum(x_dot * x_dot, axis=1, keepdims=True))
        den = jnp.maximum(nu, 1e-8) * jnp.maximum(nx, 1e-8)
        diff = u_t - x_dot
        xdot_ref[...] = x_dot
        cos_ref[...] = 1.0 - num / den
        l2_ref[...] = jnp.sum(diff * diff, axis=1, keepdims=True)


@jax.jit
def kernel(t, z, x0, x1, v0, v1, W1, b1, W2, b2):
    x = z[:, :-3]
    B, D = x.shape
    n_real = x0.shape[0] + x1.shape[0]
    ck = min(4096, max(512, n_real))
    n_pad = ((n_real + ck - 1) // ck) * ck
    qb = min(128, B)
    nchunks = n_pad // ck
    nqb = B // qb

    xk = jnp.concatenate([x0, x1], axis=0)
    vk = jnp.concatenate([v0, v1], axis=0)
    xkt = jnp.pad(xk.T, ((0, 0), (0, n_pad - n_real)))
    vkp = jnp.pad(vk, ((0, n_pad - n_real), (0, 0)))

    H = W1.shape[1]
    W1a = W1[:D]
    c1 = (t[0] * W1[D] + b1).reshape(1, H)
    b2r = b2.reshape(1, D)

    tau, h2 = pl.pallas_call(
        functools.partial(_select_kernel, n_real=n_real, n_pad=n_pad, ck=ck, qb=qb),
        grid=(nqb, nchunks),
        in_specs=[
            pl.BlockSpec((qb, D), lambda i, j: (i, 0)),
            pl.BlockSpec((D, ck), lambda i, j: (0, j)),
        ],
        out_specs=[
            pl.BlockSpec((qb, 1), lambda i, j: (i, 0)),
            pl.BlockSpec((qb, 1), lambda i, j: (i, 0)),
        ],
        out_shape=[
            jax.ShapeDtypeStruct((B, 1), jnp.float32),
            jax.ShapeDtypeStruct((B, 1), jnp.float32),
        ],
        scratch_shapes=[pltpu.VMEM((qb, n_pad), jnp.int32)],
        compiler_params=pltpu.CompilerParams(
            dimension_semantics=("parallel", "arbitrary"),
        ),
    )(x, xkt)

    xdot, cos, l2 = pl.pallas_call(
        functools.partial(_combine_kernel, n_real=n_real, n_pad=n_pad, ck=ck, qb=qb),
        grid=(nqb, nchunks),
        in_specs=[
            pl.BlockSpec((qb, D), lambda i, j: (i, 0)),
            pl.BlockSpec((D, ck), lambda i, j: (0, j)),
            pl.BlockSpec((ck, D), lambda i, j: (j, 0)),
            pl.BlockSpec((qb, 1), lambda i, j: (i, 0)),
            pl.BlockSpec((qb, 1), lambda i, j: (i, 0)),
            pl.BlockSpec((D, H), lambda i, j: (0, 0)),
            pl.BlockSpec((1, H), lambda i, j: (0, 0)),
            pl.BlockSpec((H, D), lambda i, j: (0, 0)),
            pl.BlockSpec((1, D), lambda i, j: (0, 0)),
        ],
        out_specs=[
            pl.BlockSpec((qb, D), lambda i, j: (i, 0)),
            pl.BlockSpec((qb, 1), lambda i, j: (i, 0)),
            pl.BlockSpec((qb, 1), lambda i, j: (i, 0)),
        ],
        out_shape=[
            jax.ShapeDtypeStruct((B, D), jnp.float32),
            jax.ShapeDtypeStruct((B, 1), jnp.float32),
            jax.ShapeDtypeStruct((B, 1), jnp.float32),
        ],
        scratch_shapes=[
            pltpu.VMEM((qb, D), jnp.float32),
            pltpu.VMEM((qb, 1), jnp.float32),
        ],
        compiler_params=pltpu.CompilerParams(
            dimension_semantics=("parallel", "arbitrary"),
        ),
    )(x, xkt, vkp, tau, h2, W1a, c1, W2, b2r)

    return jnp.concatenate([xdot, cos, cos, l2], axis=1)
